# baseline (speedup 1.0000x reference)
import jax
import jax.numpy as jnp
from jax import lax
from jax.experimental import pallas as pl
from jax.experimental.pallas import tpu as pltpu
from jax.experimental.pallas import tpu_sc as plsc

B, NPG, N, E = 200, 50, 10000, 320000
HID = 64
EB = 3200
W = 128
WN = 80
XO = 64

NC, NS = 2, 16
NW = NC * NS
EPW = E // NW
CH = 80
CPW = EPW // CH
NDBL = CPW // 2

NH = N // NC
DUMMY = NH
AROWS = NH + 8
EPT = E // NS
CPT = EPT // CH
ZR2 = 8
NWB = 5
NPT = NH // NWB


def _silu(v):
    return v * jax.nn.sigmoid(v)


def _sc_gather_body(t_hbm, src_hbm, dst_hbm, gs_hbm, gd_hbm,
                    idx_s, idx_d, bs0, bd0, bs1, bd1, sem, sem_wo):
    wid = lax.axis_index("c") * NS + lax.axis_index("s")
    pltpu.sync_copy(src_hbm.at[wid], idx_s)
    pltpu.sync_copy(dst_hbm.at[wid], idx_d)

    def chunk(g, bs, bd):
        hs = pltpu.async_copy(t_hbm.at[idx_s.at[g]], bs, sem)
        hd = pltpu.async_copy(t_hbm.at[idx_d.at[g]], bd, sem)
        base = wid * EPW + g * CH
        hs.wait()
        ws = pltpu.async_copy(bs, gs_hbm.at[pl.ds(base, CH)], sem_wo)
        hd.wait()
        wd = pltpu.async_copy(bd, gd_hbm.at[pl.ds(base, CH)], sem_wo)
        return ws, wd

    def dbl(i, carry):
        a, b = 2 * i, 2 * i + 1
        hs_a = pltpu.async_copy(t_hbm.at[idx_s.at[a]], bs0, sem)
        hd_a = pltpu.async_copy(t_hbm.at[idx_d.at[a]], bd0, sem)
        hs_b = pltpu.async_copy(t_hbm.at[idx_s.at[b]], bs1, sem)
        hd_b = pltpu.async_copy(t_hbm.at[idx_d.at[b]], bd1, sem)
        base_a = wid * EPW + a * CH
        base_b = wid * EPW + b * CH
        hs_a.wait()
        w0 = pltpu.async_copy(bs0, gs_hbm.at[pl.ds(base_a, CH)], sem_wo)
        hd_a.wait()
        w1 = pltpu.async_copy(bd0, gd_hbm.at[pl.ds(base_a, CH)], sem_wo)
        hs_b.wait()
        w2 = pltpu.async_copy(bs1, gs_hbm.at[pl.ds(base_b, CH)], sem_wo)
        hd_b.wait()
        w3 = pltpu.async_copy(bd1, gd_hbm.at[pl.ds(base_b, CH)], sem_wo)
        w0.wait()
        w1.wait()
        w2.wait()
        w3.wait()
        return carry

    lax.fori_loop(0, NDBL, dbl, 0)
    ws, wd = chunk(CPW - 1, bs0, bd0)
    ws.wait()
    wd.wait()


def _sc_gather(table, src3d, dst3d):
    mesh = plsc.VectorSubcoreMesh(core_axis_name="c", subcore_axis_name="s")
    f = pl.kernel(
        _sc_gather_body,
        mesh=mesh,
        out_type=[
            jax.ShapeDtypeStruct((E, W), jnp.float32),
            jax.ShapeDtypeStruct((E, W), jnp.float32),
        ],
        scratch_types=[
            pltpu.VMEM((CPW, CH), jnp.int32),
            pltpu.VMEM((CPW, CH), jnp.int32),
            pltpu.VMEM((CH, W), jnp.float32),
            pltpu.VMEM((CH, W), jnp.float32),
            pltpu.VMEM((CH, W), jnp.float32),
            pltpu.VMEM((CH, W), jnp.float32),
            pltpu.SemaphoreType.DMA,
            pltpu.SemaphoreType.DMA,
        ],
    )
    return f(table, src3d, dst3d)


def _sc_scatter_body(rows_hbm, dstl_hbm, out_hbm, idx_d, buf0, buf1, zbuf,
                     acc, sem, sem_sc):
    cid = lax.axis_index("c")
    sid = lax.axis_index("s")

    def zrow(j, carry):
        for k in range(W // 16):
            zbuf[j, 16 * k:16 * (k + 1)] = jnp.zeros((16,), jnp.float32)
        return carry

    lax.fori_loop(0, ZR2, zrow, 0)

    @pl.when(sid < NWB)
    def _zero():
        def zcopy(t, carry):
            pltpu.sync_copy(zbuf, acc.at[pl.ds(sid * NPT + t * ZR2, ZR2)])
            return carry
        lax.fori_loop(0, NPT // ZR2, zcopy, 0)

    @pl.when(sid == NWB)
    def _zero_dummy():
        pltpu.sync_copy(zbuf, acc.at[pl.ds(NH, AROWS - NH)])

    plsc.subcore_barrier()

    pltpu.sync_copy(dstl_hbm.at[cid, sid], idx_d)

    def dbl(i, carry):
        a, b = 2 * i, 2 * i + 1
        ha = pltpu.async_copy(
            rows_hbm.at[pl.ds(sid * EPT + a * CH, CH)], buf0, sem)
        hb = pltpu.async_copy(
            rows_hbm.at[pl.ds(sid * EPT + b * CH, CH)], buf1, sem)
        ha.wait()
        sa = pltpu.async_copy(buf0, acc.at[idx_d.at[a]], sem_sc, add=True)
        hb.wait()
        sb = pltpu.async_copy(buf1, acc.at[idx_d.at[b]], sem_sc, add=True)
        sa.wait()
        sb.wait()
        return carry

    lax.fori_loop(0, CPT // 2, dbl, 0)
    plsc.subcore_barrier()

    @pl.when(sid < NWB)
    def _writeback():
        pltpu.sync_copy(acc.at[pl.ds(sid * NPT, NPT)],
                        out_hbm.at[pl.ds(cid * NH + sid * NPT, NPT)])


def _sc_scatter(rows, dstl):
    mesh = plsc.VectorSubcoreMesh(core_axis_name="c", subcore_axis_name="s")
    f = pl.kernel(
        _sc_scatter_body,
        mesh=mesh,
        out_type=jax.ShapeDtypeStruct((N, W), jnp.float32),
        scratch_types=[
            pltpu.VMEM((CPT, CH), jnp.int32),
            pltpu.VMEM((CH, W), jnp.float32),
            pltpu.VMEM((CH, W), jnp.float32),
            pltpu.VMEM((ZR2, W), jnp.float32),
            pltpu.VMEM_SHARED((AROWS, W), jnp.float32),
            pltpu.SemaphoreType.DMA,
            pltpu.SemaphoreType.DMA,
        ],
    )
    return f(rows, dstl)


def _edge_kernel(gs_ref, gd_ref, e_ref, ws_ref, wd_ref, w2_ref, cw1_ref,
                 vecs_ref, out_ref):
    gs = gs_ref[...]
    gd = gd_ref[...]
    vecs = vecs_ref[...]
    g = (jnp.dot(gs[:, :HID], ws_ref[...], preferred_element_type=jnp.float32)
         + jnp.dot(gd[:, :HID], wd_ref[...],
                   preferred_element_type=jnp.float32))
    xd = gs[:, XO:XO + 8] - gd[:, XO:XO + 8]
    radial = jnp.sum(xd * xd, axis=1, keepdims=True)
    e = e_ref[...]
    pre1 = g + radial * vecs[0:1, :] + e * vecs[1:2, :] + vecs[2:3, :]
    m1 = _silu(pre1)
    m = _silu(jnp.dot(m1, w2_ref[...], preferred_element_type=jnp.float32)
              + vecs[3:4, :])
    c1 = _silu(jnp.dot(m, cw1_ref[...], preferred_element_type=jnp.float32)
               + vecs[4:5, :])
    c = jnp.sum(c1 * vecs[5:6, :], axis=1, keepdims=True)
    msg = xd * (c / (jnp.sqrt(radial) + 1e-30))
    out_ref[...] = jnp.concatenate(
        [m, msg, jnp.zeros((EB, W - HID - 8), jnp.float32)], axis=1)


def _edge_mlp(gs, gd, e_attr, ws, wd, w2, cw1, vecs):
    return pl.pallas_call(
        _edge_kernel,
        grid=(E // EB,),
        in_specs=[
            pl.BlockSpec((EB, W), lambda i: (i, 0)),
            pl.BlockSpec((EB, W), lambda i: (i, 0)),
            pl.BlockSpec((EB, 1), lambda i: (i, 0)),
            pl.BlockSpec((HID, HID), lambda i: (0, 0)),
            pl.BlockSpec((HID, HID), lambda i: (0, 0)),
            pl.BlockSpec((HID, HID), lambda i: (0, 0)),
            pl.BlockSpec((HID, HID), lambda i: (0, 0)),
            pl.BlockSpec((8, HID), lambda i: (0, 0)),
        ],
        out_specs=pl.BlockSpec((EB, W), lambda i: (i, 0)),
        out_shape=jax.ShapeDtypeStruct((E, W), jnp.float32),
    )(gs, gd, e_attr, ws, wd, w2, cw1, vecs)


def kernel(node_x, edge_index, edge_attr, sequence_data, peptide_property, params):
    src = edge_index[0].astype(jnp.int32)
    dst = edge_index[1].astype(jnp.int32)
    src3d = src.reshape(NW, CPW, CH)
    dst3d = dst.reshape(NW, CPW, CH)
    dstl = jnp.stack([
        jnp.where(dst < NH, dst, DUMMY),
        jnp.where(dst >= NH, dst - NH, DUMMY),
    ]).reshape(NC, NS, CPT, CH)
    h = jnp.pad(node_x[:, :20], ((0, 0), (0, HID - 20)))
    x = node_x[:, 20:]

    ws_l, wd_l, vecs_l, w2_l, cw1_l = [], [], [], [], []
    n1a_l, n1b_l, nb1_l, nw2_l, nb2_l = [], [], [], [], []
    for p in params['egnn']:
        ins = p['e_w1'].shape[0] // 2 - 1
        pad = HID - ins
        ws_l.append(jnp.pad(p['e_w1'][:ins], ((0, pad), (0, 0))))
        wd_l.append(jnp.pad(p['e_w1'][ins:2 * ins], ((0, pad), (0, 0))))
        vecs = jnp.stack([
            p['e_w1'][2 * ins],
            p['e_w1'][2 * ins + 1],
            p['e_b1'],
            p['e_b2'],
            p['c_b1'],
            p['c_w2'][:, 0],
        ])
        vecs_l.append(jnp.concatenate(
            [vecs, jnp.zeros((2, HID), jnp.float32)], axis=0))
        w2_l.append(p['e_w2'])
        cw1_l.append(p['c_w1'])
        n1a_l.append(jnp.pad(p['n_w1'][:ins], ((0, pad), (0, 0))))
        n1b_l.append(p['n_w1'][ins:])
        nb1_l.append(p['n_b1'])
        nw2_l.append(p['n_w2'])
        nb2_l.append(p['n_b2'])
    stacked = tuple(jnp.stack(a) for a in
                    (ws_l, wd_l, vecs_l, w2_l, cw1_l,
                     n1a_l, n1b_l, nb1_l, nw2_l, nb2_l))

    def layer(carry, wts):
        h, x = carry
        ws, wd, vecs, w2, cw1, n1a, n1b, nb1, nw2, nb2 = wts
        table = jnp.concatenate(
            [h, x, jnp.zeros((N, W - HID - 3), jnp.float32)], axis=1)
        gs, gd = _sc_gather(table, src3d, dst3d)
        out128 = _edge_mlp(gs, gd, edge_attr, ws, wd, w2, cw1, vecs)
        agg = _sc_scatter(out128, dstl)
        h_neigh = agg[:, :HID]
        x_neigh = agg[:, XO:XO + 3]
        h_out = _silu(h @ n1a + h_neigh @ n1b + nb1)
        return (h_out @ nw2 + nb2, x + x_neigh), None

    (h, x), _ = lax.scan(layer, (h, x), stacked)

    hb = h.reshape(B, NPG, HID)
    a = params['attn']
    q = hb @ a['q_w'] + a['q_b']
    k = hb @ a['k_w'] + a['k_b']
    v_ = hb @ a['v_w'] + a['v_b']
    scores = (q @ jnp.swapaxes(k, 1, 2)) / jnp.sqrt(jnp.float32(HID))
    attw = jax.nn.softmax(scores, axis=-1)
    x_gat = jnp.mean(attw @ v_, axis=1)

    v = params['vae']
    h1 = jax.nn.relu(sequence_data @ v['fc1_w'] + v['fc1_b'])
    mu = h1 @ v['fc21_w'] + v['fc21_b']
    logvar = h1 @ v['fc22_w'] + v['fc22_b']
    std = jnp.exp(0.5 * logvar)
    eps = jax.random.normal(jax.random.key(42), std.shape, jnp.float32)
    z = mu + eps * std
    pr = params['prop']
    pe = jax.nn.relu(peptide_property @ pr['w1'] + pr['b1'])
    pe = jax.nn.relu(pe @ pr['w2'] + pr['b2'])
    z_vae = jnp.concatenate([z, pe], axis=1)
    h3 = jax.nn.relu(z_vae @ v['fc3_w'] + v['fc3_b'])
    recon_x = h3 @ v['fc4_w'] + v['fc4_b']
    combined = jnp.concatenate([x_gat, z_vae, x_gat, z_vae], axis=1)
    c = params['clf']
    final_output = (jax.nn.relu(combined @ c['w1'] + c['b1']) @ c['w2']
                    + c['b2'])
    return (recon_x, mu, logvar, final_output)


# speedup vs baseline: 1.0097x; 1.0097x over previous
import jax
import jax.numpy as jnp
from jax import lax
from jax.experimental import pallas as pl
from jax.experimental.pallas import tpu as pltpu
from jax.experimental.pallas import tpu_sc as plsc

B, NPG, N, E = 200, 50, 10000, 320000
HID = 64
EB = 3200
W = 128
WN = 80
XO = 64

NC, NS = 2, 16
NW = NC * NS
EPW = E // NW
CH = 80
CPW = EPW // CH
NDBL = CPW // 2

NH = N // NC
DUMMY = NH
AROWS = NH + 8
EPT = E // NS
CPT = EPT // CH
ZR2 = 8
NWB = 5
NPT = NH // NWB


def _silu(v):
    return v * jax.nn.sigmoid(v)


def _sc_gather_body(ts_hbm, td_hbm, src_hbm, dst_hbm, gs_hbm, gd_hbm,
                    idx_s, idx_d, bs0, bd0, bs1, bd1, sem, sem_wo):
    wid = lax.axis_index("c") * NS + lax.axis_index("s")
    pltpu.sync_copy(src_hbm.at[wid], idx_s)
    pltpu.sync_copy(dst_hbm.at[wid], idx_d)

    def chunk(g, bs, bd):
        hs = pltpu.async_copy(ts_hbm.at[idx_s.at[g]], bs, sem)
        hd = pltpu.async_copy(td_hbm.at[idx_d.at[g]], bd, sem)
        base = wid * EPW + g * CH
        hs.wait()
        ws = pltpu.async_copy(bs, gs_hbm.at[pl.ds(base, CH)], sem_wo)
        hd.wait()
        wd = pltpu.async_copy(bd, gd_hbm.at[pl.ds(base, CH)], sem_wo)
        return ws, wd

    def dbl(i, carry):
        a, b = 2 * i, 2 * i + 1
        hs_a = pltpu.async_copy(ts_hbm.at[idx_s.at[a]], bs0, sem)
        hd_a = pltpu.async_copy(td_hbm.at[idx_d.at[a]], bd0, sem)
        hs_b = pltpu.async_copy(ts_hbm.at[idx_s.at[b]], bs1, sem)
        hd_b = pltpu.async_copy(td_hbm.at[idx_d.at[b]], bd1, sem)
        base_a = wid * EPW + a * CH
        base_b = wid * EPW + b * CH
        hs_a.wait()
        w0 = pltpu.async_copy(bs0, gs_hbm.at[pl.ds(base_a, CH)], sem_wo)
        hd_a.wait()
        w1 = pltpu.async_copy(bd0, gd_hbm.at[pl.ds(base_a, CH)], sem_wo)
        hs_b.wait()
        w2 = pltpu.async_copy(bs1, gs_hbm.at[pl.ds(base_b, CH)], sem_wo)
        hd_b.wait()
        w3 = pltpu.async_copy(bd1, gd_hbm.at[pl.ds(base_b, CH)], sem_wo)
        w0.wait()
        w1.wait()
        w2.wait()
        w3.wait()
        return carry

    lax.fori_loop(0, NDBL, dbl, 0)
    ws, wd = chunk(CPW - 1, bs0, bd0)
    ws.wait()
    wd.wait()


def _sc_gather(table_s, table_d, src3d, dst3d):
    mesh = plsc.VectorSubcoreMesh(core_axis_name="c", subcore_axis_name="s")
    f = pl.kernel(
        _sc_gather_body,
        mesh=mesh,
        out_type=[
            jax.ShapeDtypeStruct((E, W), jnp.float32),
            jax.ShapeDtypeStruct((E, W), jnp.float32),
        ],
        scratch_types=[
            pltpu.VMEM((CPW, CH), jnp.int32),
            pltpu.VMEM((CPW, CH), jnp.int32),
            pltpu.VMEM((CH, W), jnp.float32),
            pltpu.VMEM((CH, W), jnp.float32),
            pltpu.VMEM((CH, W), jnp.float32),
            pltpu.VMEM((CH, W), jnp.float32),
            pltpu.SemaphoreType.DMA,
            pltpu.SemaphoreType.DMA,
        ],
    )
    return f(table_s, table_d, src3d, dst3d)


def _sc_scatter_body(rows_hbm, dstl_hbm, out_hbm, idx_d, buf0, buf1, zbuf,
                     acc, sem, sem_sc):
    cid = lax.axis_index("c")
    sid = lax.axis_index("s")

    def zrow(j, carry):
        for k in range(W // 16):
            zbuf[j, 16 * k:16 * (k + 1)] = jnp.zeros((16,), jnp.float32)
        return carry

    lax.fori_loop(0, ZR2, zrow, 0)

    @pl.when(sid < NWB)
    def _zero():
        def zcopy(t, carry):
            pltpu.sync_copy(zbuf, acc.at[pl.ds(sid * NPT + t * ZR2, ZR2)])
            return carry
        lax.fori_loop(0, NPT // ZR2, zcopy, 0)

    @pl.when(sid == NWB)
    def _zero_dummy():
        pltpu.sync_copy(zbuf, acc.at[pl.ds(NH, AROWS - NH)])

    plsc.subcore_barrier()

    pltpu.sync_copy(dstl_hbm.at[cid, sid], idx_d)

    def dbl(i, carry):
        a, b = 2 * i, 2 * i + 1
        ha = pltpu.async_copy(
            rows_hbm.at[pl.ds(sid * EPT + a * CH, CH)], buf0, sem)
        hb = pltpu.async_copy(
            rows_hbm.at[pl.ds(sid * EPT + b * CH, CH)], buf1, sem)
        ha.wait()
        sa = pltpu.async_copy(buf0, acc.at[idx_d.at[a]], sem_sc, add=True)
        hb.wait()
        sb = pltpu.async_copy(buf1, acc.at[idx_d.at[b]], sem_sc, add=True)
        sa.wait()
        sb.wait()
        return carry

    lax.fori_loop(0, CPT // 2, dbl, 0)
    plsc.subcore_barrier()

    @pl.when(sid < NWB)
    def _writeback():
        pltpu.sync_copy(acc.at[pl.ds(sid * NPT, NPT)],
                        out_hbm.at[pl.ds(cid * NH + sid * NPT, NPT)])


def _sc_scatter(rows, dstl):
    mesh = plsc.VectorSubcoreMesh(core_axis_name="c", subcore_axis_name="s")
    f = pl.kernel(
        _sc_scatter_body,
        mesh=mesh,
        out_type=jax.ShapeDtypeStruct((N, W), jnp.float32),
        scratch_types=[
            pltpu.VMEM((CPT, CH), jnp.int32),
            pltpu.VMEM((CH, W), jnp.float32),
            pltpu.VMEM((CH, W), jnp.float32),
            pltpu.VMEM((ZR2, W), jnp.float32),
            pltpu.VMEM_SHARED((AROWS, W), jnp.float32),
            pltpu.SemaphoreType.DMA,
            pltpu.SemaphoreType.DMA,
        ],
    )
    return f(rows, dstl)


def _edge_kernel(gs_ref, gd_ref, e_ref, w2_ref, cw1_ref,
                 vecs_ref, out_ref):
    gs = gs_ref[...]
    gd = gd_ref[...]
    vecs = vecs_ref[...]
    g = gs[:, :HID] + gd[:, :HID]
    xd = gs[:, XO:XO + 8] - gd[:, XO:XO + 8]
    radial = jnp.sum(xd * xd, axis=1, keepdims=True)
    e = e_ref[...]
    pre1 = g + radial * vecs[0:1, :] + e * vecs[1:2, :] + vecs[2:3, :]
    m1 = _silu(pre1)
    m = _silu(jnp.dot(m1, w2_ref[...], preferred_element_type=jnp.float32)
              + vecs[3:4, :])
    c1 = _silu(jnp.dot(m, cw1_ref[...], preferred_element_type=jnp.float32)
               + vecs[4:5, :])
    c = jnp.sum(c1 * vecs[5:6, :], axis=1, keepdims=True)
    msg = xd * (c / (jnp.sqrt(radial) + 1e-30))
    out_ref[...] = jnp.concatenate(
        [m, msg, jnp.zeros((EB, W - HID - 8), jnp.float32)], axis=1)


def _edge_mlp(gs, gd, e_attr, w2, cw1, vecs):
    return pl.pallas_call(
        _edge_kernel,
        grid=(E // EB,),
        in_specs=[
            pl.BlockSpec((EB, W), lambda i: (i, 0)),
            pl.BlockSpec((EB, W), lambda i: (i, 0)),
            pl.BlockSpec((EB, 1), lambda i: (i, 0)),
            pl.BlockSpec((HID, HID), lambda i: (0, 0)),
            pl.BlockSpec((HID, HID), lambda i: (0, 0)),
            pl.BlockSpec((8, HID), lambda i: (0, 0)),
        ],
        out_specs=pl.BlockSpec((EB, W), lambda i: (i, 0)),
        out_shape=jax.ShapeDtypeStruct((E, W), jnp.float32),
    )(gs, gd, e_attr, w2, cw1, vecs)


def kernel(node_x, edge_index, edge_attr, sequence_data, peptide_property, params):
    src = edge_index[0].astype(jnp.int32)
    dst = edge_index[1].astype(jnp.int32)
    src3d = src.reshape(NW, CPW, CH)
    dst3d = dst.reshape(NW, CPW, CH)
    dstl = jnp.stack([
        jnp.where(dst < NH, dst, DUMMY),
        jnp.where(dst >= NH, dst - NH, DUMMY),
    ]).reshape(NC, NS, CPT, CH)
    h = jnp.pad(node_x[:, :20], ((0, 0), (0, HID - 20)))
    x = node_x[:, 20:]

    ws_l, wd_l, vecs_l, w2_l, cw1_l = [], [], [], [], []
    n1a_l, n1b_l, nb1_l, nw2_l, nb2_l = [], [], [], [], []
    for p in params['egnn']:
        ins = p['e_w1'].shape[0] // 2 - 1
        pad = HID - ins
        ws_l.append(jnp.pad(p['e_w1'][:ins], ((0, pad), (0, 0))))
        wd_l.append(jnp.pad(p['e_w1'][ins:2 * ins], ((0, pad), (0, 0))))
        vecs = jnp.stack([
            p['e_w1'][2 * ins],
            p['e_w1'][2 * ins + 1],
            p['e_b1'],
            p['e_b2'],
            p['c_b1'],
            p['c_w2'][:, 0],
        ])
        vecs_l.append(jnp.concatenate(
            [vecs, jnp.zeros((2, HID), jnp.float32)], axis=0))
        w2_l.append(p['e_w2'])
        cw1_l.append(p['c_w1'])
        n1a_l.append(jnp.pad(p['n_w1'][:ins], ((0, pad), (0, 0))))
        n1b_l.append(p['n_w1'][ins:])
        nb1_l.append(p['n_b1'])
        nw2_l.append(p['n_w2'])
        nb2_l.append(p['n_b2'])
    stacked = tuple(jnp.stack(a) for a in
                    (ws_l, wd_l, vecs_l, w2_l, cw1_l,
                     n1a_l, n1b_l, nb1_l, nw2_l, nb2_l))

    def layer(carry, wts):
        h, x = carry
        ws, wd, vecs, w2, cw1, n1a, n1b, nb1, nw2, nb2 = wts
        zpad = jnp.zeros((N, W - HID - 3), jnp.float32)
        table_s = jnp.concatenate([h @ ws, x, zpad], axis=1)
        table_d = jnp.concatenate([h @ wd, x, zpad], axis=1)
        gs, gd = _sc_gather(table_s, table_d, src3d, dst3d)
        out128 = _edge_mlp(gs, gd, edge_attr, w2, cw1, vecs)
        agg = _sc_scatter(out128, dstl)
        h_neigh = agg[:, :HID]
        x_neigh = agg[:, XO:XO + 3]
        h_out = _silu(h @ n1a + h_neigh @ n1b + nb1)
        return (h_out @ nw2 + nb2, x + x_neigh), None

    (h, x), _ = lax.scan(layer, (h, x), stacked)

    hb = h.reshape(B, NPG, HID)
    a = params['attn']
    q = hb @ a['q_w'] + a['q_b']
    k = hb @ a['k_w'] + a['k_b']
    v_ = hb @ a['v_w'] + a['v_b']
    scores = (q @ jnp.swapaxes(k, 1, 2)) / jnp.sqrt(jnp.float32(HID))
    attw = jax.nn.softmax(scores, axis=-1)
    x_gat = jnp.mean(attw @ v_, axis=1)

    v = params['vae']
    h1 = jax.nn.relu(sequence_data @ v['fc1_w'] + v['fc1_b'])
    mu = h1 @ v['fc21_w'] + v['fc21_b']
    logvar = h1 @ v['fc22_w'] + v['fc22_b']
    std = jnp.exp(0.5 * logvar)
    eps = jax.random.normal(jax.random.key(42), std.shape, jnp.float32)
    z = mu + eps * std
    pr = params['prop']
    pe = jax.nn.relu(peptide_property @ pr['w1'] + pr['b1'])
    pe = jax.nn.relu(pe @ pr['w2'] + pr['b2'])
    z_vae = jnp.concatenate([z, pe], axis=1)
    h3 = jax.nn.relu(z_vae @ v['fc3_w'] + v['fc3_b'])
    recon_x = h3 @ v['fc4_w'] + v['fc4_b']
    combined = jnp.concatenate([x_gat, z_vae, x_gat, z_vae], axis=1)
    c = params['clf']
    final_output = (jax.nn.relu(combined @ c['w1'] + c['b1']) @ c['w2']
                    + c['b2'])
    return (recon_x, mu, logvar, final_output)


# speedup vs baseline: 1.0234x; 1.0136x over previous
import jax
import jax.numpy as jnp
from jax import lax
from jax.experimental import pallas as pl
from jax.experimental.pallas import tpu as pltpu
from jax.experimental.pallas import tpu_sc as plsc

B, NPG, N, E = 200, 50, 10000, 320000
HID = 64
EB = 3200
W = 128
WN = 80
XO = 64

NC, NS = 2, 16
NW = NC * NS
EPW = E // NW
CH = 80
CPW = EPW // CH
NDBL = CPW // 2

NH = N // NC
DUMMY = NH
AROWS = NH + 8
EPT = E // NS
CPT = EPT // CH
ZR2 = 8
NWB = 5
NPT = NH // NWB


def _silu(v):
    return v * jax.nn.sigmoid(v)


def _row_add(dst_buf, src_buf):
    def body(r, carry):
        for k in range(W // 16):
            sl = slice(16 * k, 16 * (k + 1))
            dst_buf[r, sl] = dst_buf[r, sl] + src_buf[r, sl]
        return carry
    lax.fori_loop(0, CH, body, 0)


def _sc_gather_body(ts_hbm, td_hbm, src_hbm, dst_hbm, gsum_hbm,
                    idx_s, idx_d, bs0, bd0, bs1, bd1, sem, sem_wo):
    wid = lax.axis_index("c") * NS + lax.axis_index("s")
    pltpu.sync_copy(src_hbm.at[wid], idx_s)
    pltpu.sync_copy(dst_hbm.at[wid], idx_d)

    def dbl(i, carry):
        a, b = 2 * i, 2 * i + 1
        hs_a = pltpu.async_copy(ts_hbm.at[idx_s.at[a]], bs0, sem)
        hd_a = pltpu.async_copy(td_hbm.at[idx_d.at[a]], bd0, sem)
        hs_b = pltpu.async_copy(ts_hbm.at[idx_s.at[b]], bs1, sem)
        hd_b = pltpu.async_copy(td_hbm.at[idx_d.at[b]], bd1, sem)
        hs_a.wait()
        hd_a.wait()
        _row_add(bs0, bd0)
        w0 = pltpu.async_copy(bs0, gsum_hbm.at[pl.ds(wid * EPW + a * CH, CH)],
                              sem_wo)
        hs_b.wait()
        hd_b.wait()
        _row_add(bs1, bd1)
        w1 = pltpu.async_copy(bs1, gsum_hbm.at[pl.ds(wid * EPW + b * CH, CH)],
                              sem_wo)
        w0.wait()
        w1.wait()
        return carry

    lax.fori_loop(0, NDBL, dbl, 0)
    g = CPW - 1
    hs = pltpu.async_copy(ts_hbm.at[idx_s.at[g]], bs0, sem)
    hd = pltpu.async_copy(td_hbm.at[idx_d.at[g]], bd0, sem)
    hs.wait()
    hd.wait()
    _row_add(bs0, bd0)
    pltpu.sync_copy(bs0, gsum_hbm.at[pl.ds(wid * EPW + g * CH, CH)])


def _sc_gather(table_s, table_d, src3d, dst3d):
    mesh = plsc.VectorSubcoreMesh(core_axis_name="c", subcore_axis_name="s")
    f = pl.kernel(
        _sc_gather_body,
        mesh=mesh,
        out_type=jax.ShapeDtypeStruct((E, W), jnp.float32),
        scratch_types=[
            pltpu.VMEM((CPW, CH), jnp.int32),
            pltpu.VMEM((CPW, CH), jnp.int32),
            pltpu.VMEM((CH, W), jnp.float32),
            pltpu.VMEM((CH, W), jnp.float32),
            pltpu.VMEM((CH, W), jnp.float32),
            pltpu.VMEM((CH, W), jnp.float32),
            pltpu.SemaphoreType.DMA,
            pltpu.SemaphoreType.DMA,
        ],
    )
    return f(table_s, table_d, src3d, dst3d)


def _sc_scatter_body(rows_hbm, dstl_hbm, out_hbm, idx_d, buf0, buf1, zbuf,
                     acc, sem, sem_sc):
    cid = lax.axis_index("c")
    sid = lax.axis_index("s")

    def zrow(j, carry):
        for k in range(W // 16):
            zbuf[j, 16 * k:16 * (k + 1)] = jnp.zeros((16,), jnp.float32)
        return carry

    lax.fori_loop(0, ZR2, zrow, 0)

    @pl.when(sid < NWB)
    def _zero():
        def zcopy(t, carry):
            pltpu.sync_copy(zbuf, acc.at[pl.ds(sid * NPT + t * ZR2, ZR2)])
            return carry
        lax.fori_loop(0, NPT // ZR2, zcopy, 0)

    @pl.when(sid == NWB)
    def _zero_dummy():
        pltpu.sync_copy(zbuf, acc.at[pl.ds(NH, AROWS - NH)])

    plsc.subcore_barrier()

    pltpu.sync_copy(dstl_hbm.at[cid, sid], idx_d)

    def dbl(i, carry):
        a, b = 2 * i, 2 * i + 1
        ha = pltpu.async_copy(
            rows_hbm.at[pl.ds(sid * EPT + a * CH, CH)], buf0, sem)
        hb = pltpu.async_copy(
            rows_hbm.at[pl.ds(sid * EPT + b * CH, CH)], buf1, sem)
        ha.wait()
        sa = pltpu.async_copy(buf0, acc.at[idx_d.at[a]], sem_sc, add=True)
        hb.wait()
        sb = pltpu.async_copy(buf1, acc.at[idx_d.at[b]], sem_sc, add=True)
        sa.wait()
        sb.wait()
        return carry

    lax.fori_loop(0, CPT // 2, dbl, 0)
    plsc.subcore_barrier()

    @pl.when(sid < NWB)
    def _writeback():
        pltpu.sync_copy(acc.at[pl.ds(sid * NPT, NPT)],
                        out_hbm.at[pl.ds(cid * NH + sid * NPT, NPT)])


def _sc_scatter(rows, dstl):
    mesh = plsc.VectorSubcoreMesh(core_axis_name="c", subcore_axis_name="s")
    f = pl.kernel(
        _sc_scatter_body,
        mesh=mesh,
        out_type=jax.ShapeDtypeStruct((N, W), jnp.float32),
        scratch_types=[
            pltpu.VMEM((CPT, CH), jnp.int32),
            pltpu.VMEM((CH, W), jnp.float32),
            pltpu.VMEM((CH, W), jnp.float32),
            pltpu.VMEM((ZR2, W), jnp.float32),
            pltpu.VMEM_SHARED((AROWS, W), jnp.float32),
            pltpu.SemaphoreType.DMA,
            pltpu.SemaphoreType.DMA,
        ],
    )
    return f(rows, dstl)


def _edge_kernel(gsum_ref, e_ref, w2_ref, cw1_ref, vecs_ref, out_ref):
    gsum = gsum_ref[...]
    vecs = vecs_ref[...]
    g = gsum[:, :HID]
    xd = gsum[:, XO:XO + 8]
    radial = jnp.sum(xd * xd, axis=1, keepdims=True)
    e = e_ref[...]
    pre1 = g + radial * vecs[0:1, :] + e * vecs[1:2, :] + vecs[2:3, :]
    m1 = _silu(pre1)
    m = _silu(jnp.dot(m1, w2_ref[...], preferred_element_type=jnp.float32)
              + vecs[3:4, :])
    c1 = _silu(jnp.dot(m, cw1_ref[...], preferred_element_type=jnp.float32)
               + vecs[4:5, :])
    c = jnp.sum(c1 * vecs[5:6, :], axis=1, keepdims=True)
    msg = xd * (c / (jnp.sqrt(radial) + 1e-30))
    out_ref[...] = jnp.concatenate(
        [m, msg, jnp.zeros((EB, W - HID - 8), jnp.float32)], axis=1)


def _edge_mlp(gsum, e_attr, w2, cw1, vecs):
    return pl.pallas_call(
        _edge_kernel,
        grid=(E // EB,),
        in_specs=[
            pl.BlockSpec((EB, W), lambda i: (i, 0)),
            pl.BlockSpec((EB, 1), lambda i: (i, 0)),
            pl.BlockSpec((HID, HID), lambda i: (0, 0)),
            pl.BlockSpec((HID, HID), lambda i: (0, 0)),
            pl.BlockSpec((8, HID), lambda i: (0, 0)),
        ],
        out_specs=pl.BlockSpec((EB, W), lambda i: (i, 0)),
        out_shape=jax.ShapeDtypeStruct((E, W), jnp.float32),
    )(gsum, e_attr, w2, cw1, vecs)


def kernel(node_x, edge_index, edge_attr, sequence_data, peptide_property, params):
    src = edge_index[0].astype(jnp.int32)
    dst = edge_index[1].astype(jnp.int32)
    src3d = src.reshape(NW, CPW, CH)
    dst3d = dst.reshape(NW, CPW, CH)
    dstl = jnp.stack([
        jnp.where(dst < NH, dst, DUMMY),
        jnp.where(dst >= NH, dst - NH, DUMMY),
    ]).reshape(NC, NS, CPT, CH)
    h = jnp.pad(node_x[:, :20], ((0, 0), (0, HID - 20)))
    x = node_x[:, 20:]

    ws_l, wd_l, vecs_l, w2_l, cw1_l = [], [], [], [], []
    n1a_l, n1b_l, nb1_l, nw2_l, nb2_l = [], [], [], [], []
    for p in params['egnn']:
        ins = p['e_w1'].shape[0] // 2 - 1
        pad = HID - ins
        ws_l.append(jnp.pad(p['e_w1'][:ins], ((0, pad), (0, 0))))
        wd_l.append(jnp.pad(p['e_w1'][ins:2 * ins], ((0, pad), (0, 0))))
        vecs = jnp.stack([
            p['e_w1'][2 * ins],
            p['e_w1'][2 * ins + 1],
            p['e_b1'],
            p['e_b2'],
            p['c_b1'],
            p['c_w2'][:, 0],
        ])
        vecs_l.append(jnp.concatenate(
            [vecs, jnp.zeros((2, HID), jnp.float32)], axis=0))
        w2_l.append(p['e_w2'])
        cw1_l.append(p['c_w1'])
        n1a_l.append(jnp.pad(p['n_w1'][:ins], ((0, pad), (0, 0))))
        n1b_l.append(p['n_w1'][ins:])
        nb1_l.append(p['n_b1'])
        nw2_l.append(p['n_w2'])
        nb2_l.append(p['n_b2'])
    stacked = tuple(jnp.stack(a) for a in
                    (ws_l, wd_l, vecs_l, w2_l, cw1_l,
                     n1a_l, n1b_l, nb1_l, nw2_l, nb2_l))

    def layer(carry, wts):
        h, x = carry
        ws, wd, vecs, w2, cw1, n1a, n1b, nb1, nw2, nb2 = wts
        zpad = jnp.zeros((N, W - HID - 3), jnp.float32)
        table_s = jnp.concatenate([h @ ws, x, zpad], axis=1)
        table_d = jnp.concatenate([h @ wd, -x, zpad], axis=1)
        gsum = _sc_gather(table_s, table_d, src3d, dst3d)
        out128 = _edge_mlp(gsum, edge_attr, w2, cw1, vecs)
        agg = _sc_scatter(out128, dstl)
        h_neigh = agg[:, :HID]
        x_neigh = agg[:, XO:XO + 3]
        h_out = _silu(h @ n1a + h_neigh @ n1b + nb1)
        return (h_out @ nw2 + nb2, x + x_neigh), None

    (h, x), _ = lax.scan(layer, (h, x), stacked)

    hb = h.reshape(B, NPG, HID)
    a = params['attn']
    q = hb @ a['q_w'] + a['q_b']
    k = hb @ a['k_w'] + a['k_b']
    v_ = hb @ a['v_w'] + a['v_b']
    scores = (q @ jnp.swapaxes(k, 1, 2)) / jnp.sqrt(jnp.float32(HID))
    attw = jax.nn.softmax(scores, axis=-1)
    x_gat = jnp.mean(attw @ v_, axis=1)

    v = params['vae']
    h1 = jax.nn.relu(sequence_data @ v['fc1_w'] + v['fc1_b'])
    mu = h1 @ v['fc21_w'] + v['fc21_b']
    logvar = h1 @ v['fc22_w'] + v['fc22_b']
    std = jnp.exp(0.5 * logvar)
    eps = jax.random.normal(jax.random.key(42), std.shape, jnp.float32)
    z = mu + eps * std
    pr = params['prop']
    pe = jax.nn.relu(peptide_property @ pr['w1'] + pr['b1'])
    pe = jax.nn.relu(pe @ pr['w2'] + pr['b2'])
    z_vae = jnp.concatenate([z, pe], axis=1)
    h3 = jax.nn.relu(z_vae @ v['fc3_w'] + v['fc3_b'])
    recon_x = h3 @ v['fc4_w'] + v['fc4_b']
    combined = jnp.concatenate([x_gat, z_vae, x_gat, z_vae], axis=1)
    c = params['clf']
    final_output = (jax.nn.relu(combined @ c['w1'] + c['b1']) @ c['w2']
                    + c['b2'])
    return (recon_x, mu, logvar, final_output)


# speedup vs baseline: 1.0237x; 1.0003x over previous
import jax
import jax.numpy as jnp
from jax import lax
from jax.experimental import pallas as pl
from jax.experimental.pallas import tpu as pltpu
from jax.experimental.pallas import tpu_sc as plsc

B, NPG, N, E = 200, 50, 10000, 320000
HID = 64
EB = 3200
W = 128
XO = 64

NC, NS = 2, 16
NW = NC * NS
EPW = E // NW
CH = 80
CPW = EPW // CH
NDBL = CPW // 2

NH = N // NC
DUMMY = NH
AROWS = NH + 8
EPT = E // NS
CPT = EPT // CH
ZR2 = 8
NWB = 5
NPT = NH // NWB


def _silu(v):
    return v * jax.nn.sigmoid(v)


def _row_add(dst_buf, src_buf):
    def body(r, carry):
        for k in range(W // 16):
            sl = slice(16 * k, 16 * (k + 1))
            dst_buf[r, sl] = dst_buf[r, sl] + src_buf[r, sl]
        return carry
    lax.fori_loop(0, CH, body, 0)


def _sc_gather_body(ts_hbm, td_hbm, src_hbm, dst_hbm, gsum_hbm,
                    idx_s, idx_d, bs0, bd0, bs1, bd1, sem, sem_wo):
    wid = lax.axis_index("c") * NS + lax.axis_index("s")
    pltpu.sync_copy(src_hbm.at[wid], idx_s)
    pltpu.sync_copy(dst_hbm.at[wid], idx_d)

    def dbl(i, carry):
        a, b = 2 * i, 2 * i + 1
        hs_a = pltpu.async_copy(ts_hbm.at[idx_s.at[a]], bs0, sem)
        hd_a = pltpu.async_copy(td_hbm.at[idx_d.at[a]], bd0, sem)
        hs_b = pltpu.async_copy(ts_hbm.at[idx_s.at[b]], bs1, sem)
        hd_b = pltpu.async_copy(td_hbm.at[idx_d.at[b]], bd1, sem)
        hs_a.wait()
        hd_a.wait()
        _row_add(bs0, bd0)
        w0 = pltpu.async_copy(bs0, gsum_hbm.at[pl.ds(wid * EPW + a * CH, CH)],
                              sem_wo)
        hs_b.wait()
        hd_b.wait()
        _row_add(bs1, bd1)
        w1 = pltpu.async_copy(bs1, gsum_hbm.at[pl.ds(wid * EPW + b * CH, CH)],
                              sem_wo)
        w0.wait()
        w1.wait()
        return carry

    lax.fori_loop(0, NDBL, dbl, 0)
    g = CPW - 1
    hs = pltpu.async_copy(ts_hbm.at[idx_s.at[g]], bs0, sem)
    hd = pltpu.async_copy(td_hbm.at[idx_d.at[g]], bd0, sem)
    hs.wait()
    hd.wait()
    _row_add(bs0, bd0)
    pltpu.sync_copy(bs0, gsum_hbm.at[pl.ds(wid * EPW + g * CH, CH)])


def _sc_gather(table_s, table_d, src3d, dst3d):
    mesh = plsc.VectorSubcoreMesh(core_axis_name="c", subcore_axis_name="s")
    f = pl.kernel(
        _sc_gather_body,
        mesh=mesh,
        out_type=jax.ShapeDtypeStruct((E, W), jnp.float32),
        scratch_types=[
            pltpu.VMEM((CPW, CH), jnp.int32),
            pltpu.VMEM((CPW, CH), jnp.int32),
            pltpu.VMEM((CH, W), jnp.float32),
            pltpu.VMEM((CH, W), jnp.float32),
            pltpu.VMEM((CH, W), jnp.float32),
            pltpu.VMEM((CH, W), jnp.float32),
            pltpu.SemaphoreType.DMA,
            pltpu.SemaphoreType.DMA,
        ],
    )
    return f(table_s, table_d, src3d, dst3d)


def _sc_scatter_body(rows_hbm, dstl_hbm, out_hbm, idx_d, buf0, buf1, zbuf,
                     acc, sem, sem_sc):
    cid = lax.axis_index("c")
    sid = lax.axis_index("s")

    def zrow(j, carry):
        for k in range(W // 16):
            zbuf[j, 16 * k:16 * (k + 1)] = jnp.zeros((16,), jnp.float32)
        return carry

    lax.fori_loop(0, ZR2, zrow, 0)

    @pl.when(sid < NWB)
    def _zero():
        def zcopy(t, carry):
            pltpu.sync_copy(zbuf, acc.at[pl.ds(sid * NPT + t * ZR2, ZR2)])
            return carry
        lax.fori_loop(0, NPT // ZR2, zcopy, 0)

    @pl.when(sid == NWB)
    def _zero_dummy():
        pltpu.sync_copy(zbuf, acc.at[pl.ds(NH, AROWS - NH)])

    plsc.subcore_barrier()

    pltpu.sync_copy(dstl_hbm.at[cid, sid], idx_d)

    def dbl(i, carry):
        a, b = 2 * i, 2 * i + 1
        ha = pltpu.async_copy(
            rows_hbm.at[pl.ds(sid * EPT + a * CH, CH)], buf0, sem)
        hb = pltpu.async_copy(
            rows_hbm.at[pl.ds(sid * EPT + b * CH, CH)], buf1, sem)
        ha.wait()
        sa = pltpu.async_copy(buf0, acc.at[idx_d.at[a]], sem_sc, add=True)
        hb.wait()
        sb = pltpu.async_copy(buf1, acc.at[idx_d.at[b]], sem_sc, add=True)
        sa.wait()
        sb.wait()
        return carry

    lax.fori_loop(0, CPT // 2, dbl, 0)
    plsc.subcore_barrier()

    @pl.when(sid < NWB)
    def _writeback():
        pltpu.sync_copy(acc.at[pl.ds(sid * NPT, NPT)],
                        out_hbm.at[pl.ds(cid * NH + sid * NPT, NPT)])


def _sc_scatter(rows, dstl):
    mesh = plsc.VectorSubcoreMesh(core_axis_name="c", subcore_axis_name="s")
    f = pl.kernel(
        _sc_scatter_body,
        mesh=mesh,
        out_type=jax.ShapeDtypeStruct((N, W), jnp.float32),
        scratch_types=[
            pltpu.VMEM((CPT, CH), jnp.int32),
            pltpu.VMEM((CH, W), jnp.float32),
            pltpu.VMEM((CH, W), jnp.float32),
            pltpu.VMEM((ZR2, W), jnp.float32),
            pltpu.VMEM_SHARED((AROWS, W), jnp.float32),
            pltpu.SemaphoreType.DMA,
            pltpu.SemaphoreType.DMA,
        ],
    )
    return f(rows, dstl)


def _edge_kernel(gsum_ref, e_ref, w2_ref, cw1_ref, vecs_ref, out_ref):
    gsum = gsum_ref[...]
    vecs = vecs_ref[...]
    g = gsum[:, :HID]
    xd = gsum[:, XO:XO + 8]
    radial = jnp.sum(xd * xd, axis=1, keepdims=True)
    e = e_ref[...]
    pre1 = g + radial * vecs[0:1, :] + e * vecs[1:2, :] + vecs[2:3, :]
    m1 = _silu(pre1)
    m = _silu(jnp.dot(m1, w2_ref[...], preferred_element_type=jnp.float32)
              + vecs[3:4, :])
    c1 = _silu(jnp.dot(m, cw1_ref[...], preferred_element_type=jnp.float32)
               + vecs[4:5, :])
    c = jnp.sum(c1 * vecs[5:6, :], axis=1, keepdims=True)
    msg = xd * (c / (jnp.sqrt(radial) + 1e-30))
    out_ref[...] = jnp.concatenate(
        [m, msg, jnp.zeros((EB, W - HID - 8), jnp.float32)], axis=1)


def _edge_mlp(gsum, e_attr, w2, cw1, vecs):
    return pl.pallas_call(
        _edge_kernel,
        grid=(E // EB,),
        in_specs=[
            pl.BlockSpec((EB, W), lambda i: (i, 0)),
            pl.BlockSpec((EB, 1), lambda i: (i, 0)),
            pl.BlockSpec((HID, HID), lambda i: (0, 0)),
            pl.BlockSpec((HID, HID), lambda i: (0, 0)),
            pl.BlockSpec((8, HID), lambda i: (0, 0)),
        ],
        out_specs=pl.BlockSpec((EB, W), lambda i: (i, 0)),
        out_shape=jax.ShapeDtypeStruct((E, W), jnp.float32),
    )(gsum, e_attr, w2, cw1, vecs)


def kernel(node_x, edge_index, edge_attr, sequence_data, peptide_property, params):
    src = edge_index[0].astype(jnp.int32)
    dst = edge_index[1].astype(jnp.int32)
    src3d = src.reshape(NW, CPW, CH)
    dst3d = dst.reshape(NW, CPW, CH)
    dstl = jnp.stack([
        jnp.where(dst < NH, dst, DUMMY),
        jnp.where(dst >= NH, dst - NH, DUMMY),
    ]).reshape(NC, NS, CPT, CH)
    h = jnp.pad(node_x[:, :20], ((0, 0), (0, HID - 20)))
    x = node_x[:, 20:]

    ws_l, wd_l, vecs_l, w2_l, cw1_l = [], [], [], [], []
    n1a_l, n1b_l, nb1_l, nw2_l, nb2_l = [], [], [], [], []
    for p in params['egnn']:
        ins = p['e_w1'].shape[0] // 2 - 1
        pad = HID - ins
        ws_l.append(jnp.pad(p['e_w1'][:ins], ((0, pad), (0, 0))))
        wd_l.append(jnp.pad(p['e_w1'][ins:2 * ins], ((0, pad), (0, 0))))
        vecs = jnp.stack([
            p['e_w1'][2 * ins],
            p['e_w1'][2 * ins + 1],
            p['e_b1'],
            p['e_b2'],
            p['c_b1'],
            p['c_w2'][:, 0],
        ])
        vecs_l.append(jnp.concatenate(
            [vecs, jnp.zeros((2, HID), jnp.float32)], axis=0))
        w2_l.append(p['e_w2'])
        cw1_l.append(p['c_w1'])
        n1a_l.append(jnp.pad(p['n_w1'][:ins], ((0, pad), (0, 0))))
        n1b_l.append(p['n_w1'][ins:])
        nb1_l.append(p['n_b1'])
        nw2_l.append(p['n_w2'])
        nb2_l.append(p['n_b2'])
    stacked = tuple(jnp.stack(a) for a in
                    (ws_l, wd_l, vecs_l, w2_l, cw1_l,
                     n1a_l, n1b_l, nb1_l, nw2_l, nb2_l))

    def layer(carry, wts):
        h, x = carry
        ws, wd, vecs, w2, cw1, n1a, n1b, nb1, nw2, nb2 = wts
        zpad = jnp.zeros((N, W - HID - 3), jnp.float32)
        table_s = jnp.concatenate([h @ ws, x, zpad], axis=1)
        table_d = jnp.concatenate([h @ wd, -x, zpad], axis=1)
        gsum = _sc_gather(table_s, table_d, src3d, dst3d)
        out128 = _edge_mlp(gsum, edge_attr, w2, cw1, vecs)
        agg = _sc_scatter(out128, dstl)
        h_neigh = agg[:, :HID]
        x_neigh = agg[:, XO:XO + 3]
        h_out = _silu(h @ n1a + h_neigh @ n1b + nb1)
        return (h_out @ nw2 + nb2, x + x_neigh), None

    (h, x), _ = lax.scan(layer, (h, x), stacked)

    hb = h.reshape(B, NPG, HID)
    a = params['attn']
    q = hb @ a['q_w'] + a['q_b']
    k = hb @ a['k_w'] + a['k_b']
    v_ = hb @ a['v_w'] + a['v_b']
    scores = (q @ jnp.swapaxes(k, 1, 2)) / jnp.sqrt(jnp.float32(HID))
    attw = jax.nn.softmax(scores, axis=-1)
    x_gat = jnp.mean(attw @ v_, axis=1)

    v = params['vae']
    h1 = jax.nn.relu(sequence_data @ v['fc1_w'] + v['fc1_b'])
    mu = h1 @ v['fc21_w'] + v['fc21_b']
    logvar = h1 @ v['fc22_w'] + v['fc22_b']
    std = jnp.exp(0.5 * logvar)
    eps = jax.random.normal(jax.random.key(42), std.shape, jnp.float32)
    z = mu + eps * std
    pr = params['prop']
    pe = jax.nn.relu(peptide_property @ pr['w1'] + pr['b1'])
    pe = jax.nn.relu(pe @ pr['w2'] + pr['b2'])
    z_vae = jnp.concatenate([z, pe], axis=1)
    h3 = jax.nn.relu(z_vae @ v['fc3_w'] + v['fc3_b'])
    recon_x = h3 @ v['fc4_w'] + v['fc4_b']
    combined = jnp.concatenate([x_gat, z_vae, x_gat, z_vae], axis=1)
    c = params['clf']
    final_output = (jax.nn.relu(combined @ c['w1'] + c['b1']) @ c['w2']
                    + c['b2'])
    return (recon_x, mu, logvar, final_output)
